# Initial kernel scaffold; baseline (speedup 1.0000x reference)
import jax
import jax.numpy as jnp
from jax.experimental import pallas as pl


def kernel(output, mask, index, target):
    raise NotImplementedError("write your pallas kernel here")


# speedup vs baseline: 1.4554x; 1.4554x over previous
import functools

import jax
import jax.numpy as jnp
from jax import lax
from jax.experimental import pallas as pl
from jax.experimental.pallas import tpu as pltpu
from jax.experimental.pallas import tpu_sc as plsc

_B = 64
_C = 2
_HW = 256 * 256
_K = 500
_KPAD = 512
_NC = 2
_NS = 16
_NW = _NC * _NS
_BPW = _B // _NW
_GROWS = 16
_GCOLS = 128


def _sc_body(flat_hbm, idx_hbm, msk_hbm, tgt_hbm, num_hbm, den_hbm,
             idxb, mskb, tgtb, gidx, vals, accb, maccb, sem):
    wid = lax.axis_index("s") * _NC + lax.axis_index("c")

    for bi in range(_BPW):
        b = wid * _BPW + bi
        pltpu.sync_copy(idx_hbm.at[b], idxb.at[bi])
        pltpu.sync_copy(msk_hbm.at[b], mskb.at[bi])
        pltpu.sync_copy(tgt_hbm.at[b], tgtb.at[pl.ds(bi * 2 * _KPAD, 2 * _KPAD)])

    for r in range(_GROWS):
        bi = r // 8
        c = (r // 4) % 2
        kbase = (r % 4) * _GCOLS
        goff = (wid * _BPW + bi) * (_C * _HW) + c * _HW
        for jj in range(_GCOLS // 16):
            v = idxb[bi, pl.ds(kbase + jj * 16, 16)]
            gidx[r, pl.ds(jj * 16, 16)] = v + goff

    copies = [
        pltpu.async_copy(flat_hbm.at[gidx.at[r]], vals.at[r], sem)
        for r in range(_GROWS)
    ]
    for cp in copies:
        cp.wait()

    acc = jnp.zeros((16,), jnp.float32)
    macc = jnp.zeros((16,), jnp.float32)
    for bi in range(_BPW):
        for j in range(_KPAD // 16):
            mf = mskb[bi, pl.ds(j * 16, 16)].astype(jnp.float32)
            toff = bi * 2 * _KPAD + j * 16
            t0 = tgtb[pl.ds(toff, 16)]
            t1 = tgtb[pl.ds(toff + _KPAD, 16)]
            f0 = bi * _KPAD + j * 16
            f1 = f0 + _KPAD
            p0 = vals[f0 // _GCOLS, pl.ds(f0 % _GCOLS, 16)]
            p1 = vals[f1 // _GCOLS, pl.ds(f1 % _GCOLS, 16)]
            acc = acc + jnp.abs(p0 * mf - t0 * mf) + jnp.abs(p1 * mf - t1 * mf)
            macc = macc + mf
    accb[...] = acc
    maccb[...] = macc
    pltpu.sync_copy(accb, num_hbm.at[wid])
    pltpu.sync_copy(maccb, den_hbm.at[wid])


_sc_gather_l1 = functools.partial(
    pl.kernel,
    mesh=plsc.VectorSubcoreMesh(core_axis_name="c", subcore_axis_name="s"),
    out_type=[
        jax.ShapeDtypeStruct((_NW, 16), jnp.float32),
        jax.ShapeDtypeStruct((_NW, 16), jnp.float32),
    ],
    scratch_types=[
        pltpu.VMEM((_BPW, _KPAD), jnp.int32),
        pltpu.VMEM((_BPW, _KPAD), jnp.int32),
        pltpu.VMEM((_BPW * 2 * _KPAD,), jnp.float32),
        pltpu.VMEM((_GROWS, _GCOLS), jnp.int32),
        pltpu.VMEM((_GROWS, _GCOLS), jnp.float32),
        pltpu.VMEM((16,), jnp.float32),
        pltpu.VMEM((16,), jnp.float32),
        pltpu.SemaphoreType.DMA,
    ],
)(_sc_body)


def _finish_body(num_ref, den_ref, out_ref):
    s = jnp.sum(num_ref[...])
    d = jnp.sum(den_ref[...])
    out_ref[0, 0] = s / (d * jnp.float32(_C) + jnp.float32(1e-4))


_finish = pl.pallas_call(
    _finish_body,
    out_shape=jax.ShapeDtypeStruct((1, 1), jnp.float32),
    out_specs=pl.BlockSpec(memory_space=pltpu.SMEM),
)


def kernel(output, mask, index, target):
    B, C, H, W = output.shape
    flat = output.reshape(B * C * H * W)
    pad = _KPAD - _K
    idx_p = jnp.pad(index, ((0, 0), (0, pad)))
    msk_p = jnp.pad(mask, ((0, 0), (0, pad)))
    tgt_p = jnp.pad(target.transpose(0, 2, 1), ((0, 0), (0, 0), (0, pad)))
    tgt_p = tgt_p.reshape(B, _C * _KPAD)
    num, den = _sc_gather_l1(flat, idx_p, msk_p, tgt_p)
    loss = _finish(num, den)
    return loss[0, 0]


# speedup vs baseline: 2.7020x; 1.8566x over previous
import functools

import jax
import jax.numpy as jnp
from jax import lax
from jax.experimental import pallas as pl
from jax.experimental.pallas import tpu as pltpu
from jax.experimental.pallas import tpu_sc as plsc

_B = 64
_C = 2
_HW = 256 * 256
_K = 500
_KPAD = 512
_NC = 2
_NS = 16
_NW = _NC * _NS
_BPW = _B // _NW
_GROWS = 16
_GCOLS = 128


def _sc_body(flat_hbm, idx_hbm, msk_hbm, tgt_hbm, num_hbm, den_hbm,
             idxb, mskb, tgtb, *rest):
    gidx = rest[:_GROWS]
    vals = rest[_GROWS:2 * _GROWS]
    accb, maccb, sem = rest[2 * _GROWS:]
    wid = lax.axis_index("s") * _NC + lax.axis_index("c")

    for bi in range(_BPW):
        b = wid * _BPW + bi
        pltpu.sync_copy(idx_hbm.at[b], idxb.at[bi])
        pltpu.sync_copy(msk_hbm.at[b], mskb.at[bi])
        pltpu.sync_copy(tgt_hbm.at[b], tgtb.at[pl.ds(bi * 2 * _KPAD, 2 * _KPAD)])

    for r in range(_GROWS):
        bi = r // 8
        c = (r // 4) % 2
        kbase = (r % 4) * _GCOLS
        goff = ((wid * _BPW + bi) * _C + c) * _HW
        for jj in range(_GCOLS // 16):
            if c == 0:
                i = idxb[bi, pl.ds(kbase + jj * 16, 16)]
                v = (
                    lax.shift_left(lax.shift_right_logical(i, 11), 11)
                    + lax.shift_left(i & 128, 3)
                    + lax.shift_left(lax.shift_right_logical(i, 8) & 7, 7)
                    + (i & 127)
                )
                gidx[r][pl.ds(jj * 16, 16)] = v + goff
            else:
                v = gidx[r - 4][pl.ds(jj * 16, 16)]
                gidx[r][pl.ds(jj * 16, 16)] = v + _HW

    copies = [
        pltpu.async_copy(flat_hbm.at[gidx[r]], vals[r], sem)
        for r in range(_GROWS)
    ]
    for cp in copies:
        cp.wait()

    acc = jnp.zeros((16,), jnp.float32)
    macc = jnp.zeros((16,), jnp.float32)
    for bi in range(_BPW):
        for j in range(_KPAD // 16):
            mf = mskb[bi, pl.ds(j * 16, 16)].astype(jnp.float32)
            toff = bi * 2 * _KPAD + j * 16
            t0 = tgtb[pl.ds(toff, 16)]
            t1 = tgtb[pl.ds(toff + _KPAD, 16)]
            f0 = bi * 2 * _KPAD + j * 16
            f1 = f0 + _KPAD
            p0 = vals[f0 // _GCOLS][pl.ds(f0 % _GCOLS, 16)]
            p1 = vals[f1 // _GCOLS][pl.ds(f1 % _GCOLS, 16)]
            acc = acc + jnp.abs(p0 * mf - t0 * mf) + jnp.abs(p1 * mf - t1 * mf)
            macc = macc + mf
    accb[...] = acc
    maccb[...] = macc
    pltpu.sync_copy(accb, num_hbm.at[wid])
    pltpu.sync_copy(maccb, den_hbm.at[wid])


_sc_gather_l1 = functools.partial(
    pl.kernel,
    mesh=plsc.VectorSubcoreMesh(core_axis_name="c", subcore_axis_name="s"),
    out_type=[
        jax.ShapeDtypeStruct((_NW, 16), jnp.float32),
        jax.ShapeDtypeStruct((_NW, 16), jnp.float32),
    ],
    scratch_types=[
        pltpu.VMEM((_BPW, _KPAD), jnp.int32),
        pltpu.VMEM((_BPW, _KPAD), jnp.int32),
        pltpu.VMEM((_BPW * 2 * _KPAD,), jnp.float32),
        *[pltpu.VMEM((_GCOLS,), jnp.int32) for _ in range(_GROWS)],
        *[pltpu.VMEM((_GCOLS,), jnp.float32) for _ in range(_GROWS)],
        pltpu.VMEM((16,), jnp.float32),
        pltpu.VMEM((16,), jnp.float32),
        pltpu.SemaphoreType.DMA,
    ],
)(_sc_body)


def _finish_body(num_ref, den_ref, out_ref):
    s = jnp.sum(num_ref[...])
    d = jnp.sum(den_ref[...])
    out_ref[0, 0] = s / (d * jnp.float32(_C) + jnp.float32(1e-4))


_finish = pl.pallas_call(
    _finish_body,
    out_shape=jax.ShapeDtypeStruct((1, 1), jnp.float32),
    out_specs=pl.BlockSpec(memory_space=pltpu.SMEM),
)


def kernel(output, mask, index, target):
    B, C, H, W = output.shape
    flat = output.reshape(B, C, H // 8, 8, W // 128, 128)
    flat = flat.transpose(0, 1, 2, 4, 3, 5).reshape(B * C * H * W)
    pad = _KPAD - _K
    idx_p = jnp.pad(index, ((0, 0), (0, pad)))
    msk_p = jnp.pad(mask, ((0, 0), (0, pad)))
    tgt_p = jnp.pad(target.transpose(0, 2, 1), ((0, 0), (0, 0), (0, pad)))
    tgt_p = tgt_p.reshape(B, _C * _KPAD)
    num, den = _sc_gather_l1(flat, idx_p, msk_p, tgt_p)
    loss = _finish(num, den)
    return loss[0, 0]


# speedup vs baseline: 2.9343x; 1.0860x over previous
import functools

import jax
import jax.numpy as jnp
from jax import lax
from jax.experimental import pallas as pl
from jax.experimental.pallas import tpu as pltpu
from jax.experimental.pallas import tpu_sc as plsc

_B = 64
_C = 2
_HW = 256 * 256
_K = 500
_KPAD = 512
_NC = 2
_NS = 16
_NW = _NC * _NS
_BPW = _B // _NW
_GROWS = 16
_GCOLS = 128


def _sc_body(flat_hbm, idx_hbm, msk_hbm, tgt_hbm, num_hbm, den_hbm,
             idxb, mskb, tgtb, *rest):
    gidx = rest[:_GROWS]
    vals = rest[_GROWS:2 * _GROWS]
    accb, maccb, sem, sema, semb, semc = rest[2 * _GROWS:]
    wid = lax.axis_index("s") * _NC + lax.axis_index("c")

    idx_cp, rest_cp = [], []
    for bi in range(_BPW):
        b = wid * _BPW + bi
        idx_cp.append(pltpu.async_copy(idx_hbm.at[b], idxb.at[bi], sema))
        rest_cp.append(pltpu.async_copy(msk_hbm.at[b], mskb.at[bi], semb))
        for c in range(_C):
            rest_cp.append(pltpu.async_copy(
                tgt_hbm.at[b * _C + c],
                tgtb.at[pl.ds((bi * _C + c) * _KPAD, _KPAD)], semc))
    for cp in idx_cp:
        cp.wait()

    for r in range(_GROWS):
        bi = r // 8
        c = (r // 4) % 2
        kbase = (r % 4) * _GCOLS
        goff = ((wid * _BPW + bi) * _C + c) * _HW
        for jj in range(_GCOLS // 16):
            if c == 0:
                i = idxb[bi, pl.ds(kbase + jj * 16, 16)]
                v = (
                    lax.shift_left(lax.shift_right_logical(i, 11), 11)
                    + lax.shift_left(i & 128, 3)
                    + lax.shift_left(lax.shift_right_logical(i, 8) & 7, 7)
                    + (i & 127)
                )
                gidx[r][pl.ds(jj * 16, 16)] = v + goff
            else:
                v = gidx[r - 4][pl.ds(jj * 16, 16)]
                gidx[r][pl.ds(jj * 16, 16)] = v + _HW

    copies = [
        pltpu.async_copy(flat_hbm.at[gidx[r]], vals[r], sem)
        for r in range(_GROWS)
    ]
    for cp in rest_cp:
        cp.wait()
    for cp in copies:
        cp.wait()

    acc = jnp.zeros((16,), jnp.float32)
    macc = jnp.zeros((16,), jnp.float32)
    for bi in range(_BPW):
        for j in range(_KPAD // 16):
            mf = mskb[bi, pl.ds(j * 16, 16)].astype(jnp.float32)
            toff = bi * 2 * _KPAD + j * 16
            t0 = tgtb[pl.ds(toff, 16)]
            t1 = tgtb[pl.ds(toff + _KPAD, 16)]
            f0 = bi * 2 * _KPAD + j * 16
            f1 = f0 + _KPAD
            p0 = vals[f0 // _GCOLS][pl.ds(f0 % _GCOLS, 16)]
            p1 = vals[f1 // _GCOLS][pl.ds(f1 % _GCOLS, 16)]
            acc = acc + jnp.abs(p0 * mf - t0 * mf) + jnp.abs(p1 * mf - t1 * mf)
            macc = macc + mf
    accb[...] = acc
    maccb[...] = macc
    pltpu.sync_copy(accb, num_hbm.at[wid])
    pltpu.sync_copy(maccb, den_hbm.at[wid])


_sc_gather_l1 = functools.partial(
    pl.kernel,
    mesh=plsc.VectorSubcoreMesh(core_axis_name="c", subcore_axis_name="s"),
    out_type=[
        jax.ShapeDtypeStruct((_NW, 16), jnp.float32),
        jax.ShapeDtypeStruct((_NW, 16), jnp.float32),
    ],
    scratch_types=[
        pltpu.VMEM((_BPW, _KPAD), jnp.int32),
        pltpu.VMEM((_BPW, _KPAD), jnp.int32),
        pltpu.VMEM((_BPW * 2 * _KPAD,), jnp.float32),
        *[pltpu.VMEM((_GCOLS,), jnp.int32) for _ in range(_GROWS)],
        *[pltpu.VMEM((_GCOLS,), jnp.float32) for _ in range(_GROWS)],
        pltpu.VMEM((16,), jnp.float32),
        pltpu.VMEM((16,), jnp.float32),
        pltpu.SemaphoreType.DMA,
        pltpu.SemaphoreType.DMA,
        pltpu.SemaphoreType.DMA,
        pltpu.SemaphoreType.DMA,
    ],
)(_sc_body)


def _finish_body(num_ref, den_ref, out_ref):
    s = jnp.sum(num_ref[...])
    d = jnp.sum(den_ref[...])
    out_ref[0, 0] = s / (d * jnp.float32(_C) + jnp.float32(1e-4))


_finish = pl.pallas_call(
    _finish_body,
    out_shape=jax.ShapeDtypeStruct((1, 1), jnp.float32),
    out_specs=pl.BlockSpec(memory_space=pltpu.SMEM),
)


def kernel(output, mask, index, target):
    B, C, H, W = output.shape
    flat = output.reshape(B, C, H // 8, 8, W // 128, 128)
    flat = flat.transpose(0, 1, 2, 4, 3, 5).reshape(B * C * H * W)
    pad = _KPAD - _K
    idx_p = jnp.pad(index, ((0, 0), (0, pad)))
    msk_p = jnp.pad(mask, ((0, 0), (0, pad)))
    tgt_p = jnp.pad(target.transpose(0, 2, 1).reshape(B * C, _K),
                    ((0, 0), (0, pad)))
    num, den = _sc_gather_l1(flat, idx_p, msk_p, tgt_p)
    loss = _finish(num, den)
    return loss[0, 0]


# speedup vs baseline: 2.9345x; 1.0001x over previous
import functools

import jax
import jax.numpy as jnp
from jax import lax
from jax.experimental import pallas as pl
from jax.experimental.pallas import tpu as pltpu
from jax.experimental.pallas import tpu_sc as plsc

_B = 64
_C = 2
_HW = 256 * 256
_K = 500
_KPAD = 512
_NC = 2
_NS = 16
_NW = _NC * _NS
_BPW = _B // _NW
_GROWS = 16
_GCOLS = 128


def _sc_body(flat_hbm, idx_hbm, msk_hbm, tgt_hbm, out_hbm,
             idxb, mskb, tgtb, *rest):
    gidx = rest[:_GROWS]
    vals = rest[_GROWS:2 * _GROWS]
    accb, sem, sema, semb, semc = rest[2 * _GROWS:]
    wid = lax.axis_index("s") * _NC + lax.axis_index("c")

    idx_cp, rest_cp = [], []
    for bi in range(_BPW):
        b = wid * _BPW + bi
        idx_cp.append(pltpu.async_copy(idx_hbm.at[b], idxb.at[bi], sema))
        rest_cp.append(pltpu.async_copy(msk_hbm.at[b], mskb.at[bi], semb))
        for c in range(_C):
            rest_cp.append(pltpu.async_copy(
                tgt_hbm.at[b * _C + c],
                tgtb.at[pl.ds((bi * _C + c) * _KPAD, _KPAD)], semc))
    for cp in idx_cp:
        cp.wait()

    for r in range(_GROWS):
        bi = r // 8
        c = (r // 4) % 2
        kbase = (r % 4) * _GCOLS
        goff = ((wid * _BPW + bi) * _C + c) * _HW
        for jj in range(_GCOLS // 16):
            if c == 0:
                i = idxb[bi, pl.ds(kbase + jj * 16, 16)]
                v = (
                    lax.shift_left(lax.shift_right_logical(i, 11), 11)
                    + lax.shift_left(i & 128, 3)
                    + lax.shift_left(lax.shift_right_logical(i, 8) & 7, 7)
                    + (i & 127)
                )
                gidx[r][pl.ds(jj * 16, 16)] = v + goff
            else:
                v = gidx[r - 4][pl.ds(jj * 16, 16)]
                gidx[r][pl.ds(jj * 16, 16)] = v + _HW

    copies = [
        pltpu.async_copy(flat_hbm.at[gidx[r]], vals[r], sem)
        for r in range(_GROWS)
    ]
    for cp in rest_cp:
        cp.wait()
    for cp in copies:
        cp.wait()

    acc = jnp.zeros((16,), jnp.float32)
    macc = jnp.zeros((16,), jnp.float32)
    for bi in range(_BPW):
        for j in range(_KPAD // 16):
            mf = mskb[bi, pl.ds(j * 16, 16)].astype(jnp.float32)
            toff = bi * 2 * _KPAD + j * 16
            t0 = tgtb[pl.ds(toff, 16)]
            t1 = tgtb[pl.ds(toff + _KPAD, 16)]
            f0 = bi * 2 * _KPAD + j * 16
            f1 = f0 + _KPAD
            p0 = vals[f0 // _GCOLS][pl.ds(f0 % _GCOLS, 16)]
            p1 = vals[f1 // _GCOLS][pl.ds(f1 % _GCOLS, 16)]
            acc = acc + jnp.abs(p0 * mf - t0 * mf) + jnp.abs(p1 * mf - t1 * mf)
            macc = macc + mf
    accb[pl.ds(0, 16)] = acc
    accb[pl.ds(16, 16)] = macc
    pltpu.sync_copy(accb, out_hbm.at[wid])


_sc_gather_l1 = functools.partial(
    pl.kernel,
    mesh=plsc.VectorSubcoreMesh(core_axis_name="c", subcore_axis_name="s"),
    out_type=jax.ShapeDtypeStruct((_NW, 32), jnp.float32),
    scratch_types=[
        pltpu.VMEM((_BPW, _KPAD), jnp.int32),
        pltpu.VMEM((_BPW, _KPAD), jnp.int32),
        pltpu.VMEM((_BPW * 2 * _KPAD,), jnp.float32),
        *[pltpu.VMEM((_GCOLS,), jnp.int32) for _ in range(_GROWS)],
        *[pltpu.VMEM((_GCOLS,), jnp.float32) for _ in range(_GROWS)],
        pltpu.VMEM((32,), jnp.float32),
        pltpu.SemaphoreType.DMA,
        pltpu.SemaphoreType.DMA,
        pltpu.SemaphoreType.DMA,
        pltpu.SemaphoreType.DMA,
    ],
)(_sc_body)


def _finish_body(part_ref, out_ref):
    s = jnp.sum(part_ref[:, :16])
    d = jnp.sum(part_ref[:, 16:])
    out_ref[0, 0] = s / (d * jnp.float32(_C) + jnp.float32(1e-4))


_finish = pl.pallas_call(
    _finish_body,
    out_shape=jax.ShapeDtypeStruct((1, 1), jnp.float32),
    out_specs=pl.BlockSpec(memory_space=pltpu.SMEM),
)


def kernel(output, mask, index, target):
    B, C, H, W = output.shape
    flat = output.reshape(B, C, H // 8, 8, W // 128, 128)
    flat = flat.transpose(0, 1, 2, 4, 3, 5).reshape(B * C * H * W)
    pad = _KPAD - _K
    idx_p = jnp.pad(index, ((0, 0), (0, pad)))
    msk_p = jnp.pad(mask, ((0, 0), (0, pad)))
    tgt_p = jnp.pad(target.transpose(0, 2, 1).reshape(B * C, _K),
                    ((0, 0), (0, pad)))
    parts = _sc_gather_l1(flat, idx_p, msk_p, tgt_p)
    loss = _finish(parts)
    return loss[0, 0]
